# Initial kernel scaffold; baseline (speedup 1.0000x reference)
import jax
import jax.numpy as jnp
from jax.experimental import pallas as pl


def kernel(x, edge_index, batch, edge_weight, W_rel1, b_rel1, W_root1, W_rel2, b_rel2, W_root2, W_rel3, b_rel3, W_root3, W_rel4, b_rel4, W_root4, W_rel5, b_rel5, W_root5, W_fc1, b_fc1, W_fc2, b_fc2):
    raise NotImplementedError("write your pallas kernel here")


# speedup vs baseline: 10.9979x; 10.9979x over previous
import functools

import jax
import jax.numpy as jnp
from jax import lax
from jax.experimental import pallas as pl
from jax.experimental.pallas import tpu as pltpu
from jax.experimental.pallas import tpu_sc as plsc

_N = 10000
_E = 320000
_DIM = 32
_G = 128

_CB = 128
_NSUB = 16
_NCORE = 2
_NW = _NCORE * _NSUB
_CPW = 80
_NCHUNK = _CPW * _NW
_EPAD = _NCHUNK * _CB
_NPAD = 10112
_RPS = _NPAD // _NSUB

_BLK = 2000
_NB = _N // _BLK


def _sc_scatter_body(p_hbm, src_hbm, dst_hbm, ew_hbm, out_hbm,
                     acc, src_v, dst_v, ew_v, rows, zbuf, gsem):
    c = lax.axis_index("c")
    s = lax.axis_index("s")
    w = c * _NSUB + s

    pltpu.sync_copy(src_hbm.at[pl.ds(w * _CPW, _CPW)], src_v)
    pltpu.sync_copy(dst_hbm.at[pl.ds(w * _CPW, _CPW)], dst_v)
    pltpu.sync_copy(ew_hbm.at[pl.ds(w * _CPW, _CPW)], ew_v)

    zv = jnp.zeros((16,), jnp.float32)

    def zrow(i, carry):
        zbuf[i, pl.ds(0, 16)] = zv
        zbuf[i, pl.ds(16, 16)] = zv
        return carry

    lax.fori_loop(0, _RPS, zrow, 0)
    pltpu.sync_copy(zbuf, acc.at[pl.ds(s * _RPS, _RPS)])
    plsc.subcore_barrier()

    def chunk(k, carry):
        pltpu.async_copy(p_hbm.at[src_v.at[k]], rows, gsem).wait()

        def edges(g, carry2):
            ewv = ew_v[k, pl.ds(g * 16, 16)]
            base = g * 16
            for u in range(16):
                e = base + u
                wv = jnp.full((16,), ewv[u], jnp.float32)
                rows[e, pl.ds(0, 16)] = rows[e, pl.ds(0, 16)] * wv
                rows[e, pl.ds(16, 16)] = rows[e, pl.ds(16, 16)] * wv
            return carry2

        lax.fori_loop(0, _CB // 16, edges, 0)

        pltpu.sync_copy(rows, acc.at[dst_v.at[k]], add=True)
        return carry

    lax.fori_loop(0, _CPW, chunk, 0)

    plsc.subcore_barrier()
    pltpu.sync_copy(acc.at[pl.ds(s * _RPS, _RPS)],
                    out_hbm.at[c, pl.ds(s * _RPS, _RPS)])


@functools.partial(
    pl.kernel,
    out_type=jax.ShapeDtypeStruct((_NCORE, _NPAD, _DIM), jnp.float32),
    mesh=plsc.VectorSubcoreMesh(core_axis_name="c", subcore_axis_name="s"),
    scratch_types=[
        pltpu.VMEM_SHARED((_NPAD, _DIM), jnp.float32),
        pltpu.VMEM((_CPW, _CB), jnp.int32),
        pltpu.VMEM((_CPW, _CB), jnp.int32),
        pltpu.VMEM((_CPW, _CB), jnp.float32),
        pltpu.VMEM((_CB, _DIM), jnp.float32),
        pltpu.VMEM((_RPS, _DIM), jnp.float32),
        pltpu.SemaphoreType.DMA,
    ],
    compiler_params=pltpu.CompilerParams(use_tc_tiling_on_sc=False),
)
def _sc_scatter(p_hbm, src_hbm, dst_hbm, ew_hbm, out_hbm,
                acc, src_v, dst_v, ew_v, rows, zbuf, gsem):
    _sc_scatter_body(p_hbm, src_hbm, dst_hbm, ew_hbm, out_hbm,
                     acc, src_v, dst_v, ew_v, rows, zbuf, gsem)


def _mm_body(x_ref, w_ref, o_ref):
    o_ref[...] = jnp.dot(x_ref[...], w_ref[...],
                         preferred_element_type=jnp.float32)


def _mm(x, w):
    di, do = w.shape
    return pl.pallas_call(
        _mm_body,
        grid=(_NB,),
        in_specs=[pl.BlockSpec((_BLK, di), lambda i: (i, 0)),
                  pl.BlockSpec((di, do), lambda i: (0, 0))],
        out_specs=pl.BlockSpec((_BLK, do), lambda i: (i, 0)),
        out_shape=jax.ShapeDtypeStruct((_N, do), jnp.float32),
    )(x, w)


def _combine_body2(parts_ref, hprev_ref, wt_ref, br_ref, wn_ref,
                   h_ref, p_ref):
    agg = parts_ref[0] + parts_ref[1]
    h = jnp.maximum(
        agg + br_ref[...]
        + jnp.dot(hprev_ref[...], wt_ref[...],
                  preferred_element_type=jnp.float32), 0.0)
    h_ref[...] = h
    p_ref[...] = jnp.dot(h, wn_ref[...], preferred_element_type=jnp.float32)


def _combine_body1(parts_ref, hprev_ref, wt_ref, br_ref, h_ref):
    agg = parts_ref[0] + parts_ref[1]
    h_ref[...] = jnp.maximum(
        agg + br_ref[...]
        + jnp.dot(hprev_ref[...], wt_ref[...],
                  preferred_element_type=jnp.float32), 0.0)


def _combine(parts, hprev, wt, br, wn):
    di = hprev.shape[1]
    parts_spec = pl.BlockSpec((_NCORE, _BLK, _DIM), lambda i: (0, i, 0))
    hprev_spec = pl.BlockSpec((_BLK, di), lambda i: (i, 0))
    wt_spec = pl.BlockSpec((di, _DIM), lambda i: (0, 0))
    br_spec = pl.BlockSpec((1, _DIM), lambda i: (0, 0))
    out_spec = pl.BlockSpec((_BLK, _DIM), lambda i: (i, 0))
    h_shape = jax.ShapeDtypeStruct((_N, _DIM), jnp.float32)
    if wn is not None:
        return pl.pallas_call(
            _combine_body2,
            grid=(_NB,),
            in_specs=[parts_spec, hprev_spec, wt_spec, br_spec,
                      pl.BlockSpec((_DIM, _DIM), lambda i: (0, 0))],
            out_specs=(out_spec, out_spec),
            out_shape=(h_shape, h_shape),
        )(parts, hprev, wt, br, wn)
    return pl.pallas_call(
        _combine_body1,
        grid=(_NB,),
        in_specs=[parts_spec, hprev_spec, wt_spec, br_spec],
        out_specs=out_spec,
        out_shape=h_shape,
    )(parts, hprev, wt, br)


def _pool_head_body(b_ref, h1, h2, h3, h4, h5,
                    w1_ref, b1_ref, w2_ref, b2_ref, o_ref, acc_ref):
    i = pl.program_id(0)

    @pl.when(i == 0)
    def _():
        acc_ref[...] = jnp.zeros_like(acc_ref)

    ids = b_ref[...]
    gi = lax.broadcasted_iota(jnp.int32, (_BLK, _G), 1)
    onehot = jnp.where(ids == gi, 1.0, 0.0).astype(jnp.float32)
    for l, hr in enumerate((h1, h2, h3, h4, h5)):
        contrib = lax.dot_general(onehot, hr[...],
                                  (((0,), (0,)), ((), ())),
                                  preferred_element_type=jnp.float32)
        acc_ref[:, l * _DIM:(l + 1) * _DIM] += contrib

    @pl.when(i == _NB - 1)
    def _():
        pooled = acc_ref[...]
        hfc = jnp.maximum(
            jnp.dot(pooled, w1_ref[...],
                    preferred_element_type=jnp.float32) + b1_ref[...], 0.0)
        logits = jnp.dot(hfc, w2_ref[...],
                         preferred_element_type=jnp.float32) + b2_ref[...]
        m = jnp.max(logits, axis=-1, keepdims=True)
        lse = m + jnp.log(jnp.sum(jnp.exp(logits - m), axis=-1,
                                  keepdims=True))
        o_ref[...] = logits - lse


def _pool_head(batch2d, hs, w1, b1, w2, b2):
    h_spec = pl.BlockSpec((_BLK, _DIM), lambda i: (i, 0))
    return pl.pallas_call(
        _pool_head_body,
        grid=(_NB,),
        in_specs=[pl.BlockSpec((_BLK, 1), lambda i: (i, 0))]
        + [h_spec] * 5
        + [pl.BlockSpec((5 * _DIM, _DIM), lambda i: (0, 0)),
           pl.BlockSpec((1, _DIM), lambda i: (0, 0)),
           pl.BlockSpec((_DIM, 2), lambda i: (0, 0)),
           pl.BlockSpec((1, 2), lambda i: (0, 0))],
        out_specs=pl.BlockSpec((_G, 2), lambda i: (0, 0)),
        out_shape=jax.ShapeDtypeStruct((_G, 2), jnp.float32),
        scratch_shapes=[pltpu.VMEM((_G, 5 * _DIM), jnp.float32)],
    )(batch2d, *hs, w1, b1, w2, b2)


def kernel(x, edge_index, batch, edge_weight,
           W_rel1, b_rel1, W_root1, W_rel2, b_rel2, W_root2,
           W_rel3, b_rel3, W_root3, W_rel4, b_rel4, W_root4,
           W_rel5, b_rel5, W_root5, W_fc1, b_fc1, W_fc2, b_fc2):
    pad = _EPAD - _E
    fill = (jnp.arange(pad, dtype=jnp.int32) * 97) % _N
    src2d = jnp.concatenate([edge_index[0], fill]).reshape(_NCHUNK, _CB)
    dst2d = jnp.concatenate([edge_index[1], fill]).reshape(_NCHUNK, _CB)
    ew2d = jnp.concatenate(
        [edge_weight, jnp.zeros((pad,), jnp.float32)]).reshape(_NCHUNK, _CB)
    batch2d = batch.reshape(_N, 1)

    layers = [(W_rel1, b_rel1, W_root1), (W_rel2, b_rel2, W_root2),
              (W_rel3, b_rel3, W_root3), (W_rel4, b_rel4, W_root4),
              (W_rel5, b_rel5, W_root5)]

    h = x
    p = _mm(x, W_rel1)
    hs = []
    for l in range(5):
        _, br, wt = layers[l]
        parts = _sc_scatter(p, src2d, dst2d, ew2d)
        wn = layers[l + 1][0] if l < 4 else None
        if wn is not None:
            h, p = _combine(parts, h, wt, br.reshape(1, _DIM), wn)
        else:
            h = _combine(parts, h, wt, br.reshape(1, _DIM), None)
        hs.append(h)

    return _pool_head(batch2d, hs, W_fc1, b_fc1.reshape(1, _DIM),
                      W_fc2, b_fc2.reshape(1, 2))
